# baseline (speedup 1.0000x reference)
import functools

import jax
import jax.numpy as jnp
from jax import lax
from jax.experimental import pallas as pl
from jax.experimental.pallas import tpu as pltpu
from jax.experimental.pallas import tpu_sc as plsc

_LANES = 16
_LANE = 128
_G = 7


@functools.cache
def _build_sc_lookup(B, K, N):
    info = plsc.get_sparse_core_info()
    NC, NS = info.num_cores, info.num_subcores
    NW = NC * NS
    BPW = B // NW
    assert B % (8 * NW) == 0 and K % _LANES == 0
    groups = []
    b0 = 0
    while b0 < BPW:
        groups.append((b0, min(_G, BPW - b0)))
        b0 += _G

    mesh = plsc.VectorSubcoreMesh(core_axis_name="c", subcore_axis_name="s")

    @functools.partial(
        pl.kernel,
        out_type=(
            jax.ShapeDtypeStruct((B, K), jnp.float32),
            jax.ShapeDtypeStruct((B,), jnp.float32),
        ),
        mesh=mesh,
        compiler_params=pltpu.CompilerParams(needs_layout_passes=False),
        scratch_types=[
            pltpu.VMEM((BPW,), jnp.int32),
            pltpu.VMEM((2, _G, K, _LANE), jnp.float32),
            pltpu.VMEM((1 + BPW, _LANE), jnp.float32),
            pltpu.VMEM((BPW, K), jnp.float32),
            pltpu.VMEM((BPW,), jnp.float32),
            pltpu.SemaphoreType.DMA,
            pltpu.SemaphoreType.DMA,
            pltpu.SemaphoreType.DMA,
        ],
    )
    def lookup(idx_hbm, wpos_hbm, whet_hbm, pos_out, het_out,
               idx_v, blk_v, hblk_v, vals_v, het_v, psem0, psem1, hsem):
        wid = lax.axis_index("s") * NC + lax.axis_index("c")
        base = wid * BPW
        pltpu.sync_copy(idx_hbm.at[pl.ds(base, BPW)], idx_v)

        lanes = lax.iota(jnp.int32, _LANES)
        psems = (psem0, psem1)
        NG = len(groups)

        def fire(g):
            buf = g % 2
            gb, gn = groups[g]
            cps = []
            for s in range(gn):
                b = gb + s
                n = idx_v[pl.ds((b // 16) * 16, 16)][b % 16]
                toff = pl.multiple_of((n // _LANE) * _LANE, _LANE)
                cps.append(pltpu.async_copy(
                    wpos_hbm.at[:, pl.ds(toff, _LANE)],
                    blk_v.at[buf, s], psems[buf]))
            return cps

        def fire_het(g):
            gb, gn = groups[g]
            cps = []
            for s in range(gn):
                b = gb + s
                n = idx_v[pl.ds((b // 16) * 16, 16)][b % 16]
                toff = pl.multiple_of((n // _LANE) * _LANE, _LANE)
                cps.append(pltpu.async_copy(
                    whet_hbm.at[:, pl.ds(toff, _LANE)],
                    hblk_v.at[pl.ds(1 + b, 1), :], hsem))
            return cps

        inflight = {0: fire(0)}
        hcopies = []
        for g in range(NG):
            if g + 1 < NG:
                inflight[g + 1] = fire(g + 1)
            hcopies.extend(fire_het(g))
            for cp in inflight.pop(g):
                cp.wait()
            buf = g % 2
            gb, gn = groups[g]
            for s in range(gn):
                b = gb + s
                n = idx_v[pl.ds((b // 16) * 16, 16)][b % 16]
                lvec = jnp.full((_LANES,), n % _LANE, jnp.int32)
                blk = blk_v.at[buf, s]
                for k0 in range(0, K, _LANES):
                    vals_v[b, pl.ds(k0, _LANES)] = plsc.load_gather(
                        blk, [k0 + lanes, lvec])

        for cp in hcopies:
            cp.wait()
        for h in range(BPW // _LANES):
            rvec = 1 + h * _LANES + lanes
            lvec = idx_v[pl.ds(h * _LANES, _LANES)] % _LANE
            het_v[pl.ds(h * _LANES, _LANES)] = plsc.load_gather(
                hblk_v, [rvec, lvec])

        pltpu.sync_copy(vals_v, pos_out.at[pl.ds(base, BPW), :])
        pltpu.sync_copy(het_v, het_out.at[pl.ds(base, BPW)])

    return lookup


def kernel(indices, W_pos, W_het):
    K, N = W_pos.shape
    B = indices.shape[0]
    lookup = _build_sc_lookup(B, K, N)
    pos, het = lookup(indices.astype(jnp.int32), W_pos, W_het)
    return pos, het.reshape(B, 1)


# speedup vs baseline: 1.0129x; 1.0129x over previous
import functools

import jax
import jax.numpy as jnp
from jax import lax
from jax.experimental import pallas as pl
from jax.experimental.pallas import tpu as pltpu
from jax.experimental.pallas import tpu_sc as plsc

_LANES = 16
_LANE = 128
_G = 7


@functools.cache
def _build_sc_lookup(B, K, N):
    info = plsc.get_sparse_core_info()
    NC, NS = info.num_cores, info.num_subcores
    NW = NC * NS
    BPW = B // NW
    assert B % (8 * NW) == 0 and K % _LANES == 0
    groups = []
    b0 = 0
    while b0 < BPW:
        groups.append((b0, min(_G, BPW - b0)))
        b0 += _G

    mesh = plsc.VectorSubcoreMesh(core_axis_name="c", subcore_axis_name="s")

    @functools.partial(
        pl.kernel,
        out_type=(
            jax.ShapeDtypeStruct((B, K), jnp.float32),
            jax.ShapeDtypeStruct((B,), jnp.float32),
        ),
        mesh=mesh,
        compiler_params=pltpu.CompilerParams(needs_layout_passes=False),
        scratch_types=[
            pltpu.VMEM((BPW,), jnp.int32),
            pltpu.VMEM((2, _G, K, _LANE), jnp.float32),
            pltpu.VMEM((1 + BPW, _LANE), jnp.float32),
            pltpu.VMEM((BPW, K), jnp.float32),
            pltpu.VMEM((BPW,), jnp.float32),
            pltpu.SemaphoreType.DMA,
            pltpu.SemaphoreType.DMA,
            pltpu.SemaphoreType.DMA,
        ],
    )
    def lookup(idx_hbm, wpos_hbm, whet_hbm, pos_out, het_out,
               idx_v, blk_v, hblk_v, vals_v, het_v, psem0, psem1, hsem):
        wid = lax.axis_index("s") * NC + lax.axis_index("c")
        base = wid * BPW
        pltpu.sync_copy(idx_hbm.at[pl.ds(base, BPW)], idx_v)

        lanes = lax.iota(jnp.int32, _LANES)
        psems = (psem0, psem1)
        NG = len(groups)

        def fire(g):
            buf = g % 2
            gb, gn = groups[g]
            cps = []
            for s in range(gn):
                b = gb + s
                n = idx_v[pl.ds((b // 16) * 16, 16)][b % 16]
                toff = pl.multiple_of((n // _LANE) * _LANE, _LANE)
                cps.append(pltpu.async_copy(
                    wpos_hbm.at[:, pl.ds(toff, _LANE)],
                    blk_v.at[buf, s], psems[buf]))
            return cps

        inflight = {0: fire(0)}
        hcopies = []
        for b in range(BPW):
            n = idx_v[pl.ds((b // 16) * 16, 16)][b % 16]
            toff = pl.multiple_of((n // _LANE) * _LANE, _LANE)
            hcopies.append(pltpu.async_copy(
                whet_hbm.at[:, pl.ds(toff, _LANE)],
                hblk_v.at[pl.ds(1 + b, 1), :], hsem))

        for g in range(NG):
            if g + 1 < NG:
                inflight[g + 1] = fire(g + 1)
            for cp in inflight.pop(g):
                cp.wait()
            buf = g % 2
            gb, gn = groups[g]
            for s in range(gn):
                b = gb + s
                n = idx_v[pl.ds((b // 16) * 16, 16)][b % 16]
                lvec = jnp.full((_LANES,), n % _LANE, jnp.int32)
                blk = blk_v.at[buf, s]
                for k0 in range(0, K, _LANES):
                    vals_v[b, pl.ds(k0, _LANES)] = plsc.load_gather(
                        blk, [k0 + lanes, lvec])

        for cp in hcopies:
            cp.wait()
        for h in range(BPW // _LANES):
            rvec = 1 + h * _LANES + lanes
            lvec = idx_v[pl.ds(h * _LANES, _LANES)] % _LANE
            het_v[pl.ds(h * _LANES, _LANES)] = plsc.load_gather(
                hblk_v, [rvec, lvec])

        pltpu.sync_copy(vals_v, pos_out.at[pl.ds(base, BPW), :])
        pltpu.sync_copy(het_v, het_out.at[pl.ds(base, BPW)])

    return lookup


def kernel(indices, W_pos, W_het):
    K, N = W_pos.shape
    B = indices.shape[0]
    lookup = _build_sc_lookup(B, K, N)
    pos, het = lookup(indices.astype(jnp.int32), W_pos, W_het)
    return pos, het.reshape(B, 1)
